# baseline (speedup 1.0000x reference)
import functools

import jax
import jax.numpy as jnp
from jax import lax
from jax.experimental import pallas as pl
from jax.experimental.pallas import tpu as pltpu
from jax.experimental.pallas import tpu_sc as plsc

N = 100000
D = 576
DPAD = 640
S = 4
D_K = 64
S1, S2, S3 = 256, 512, 576
K = 4096
D_B = 64
D_A = 64
R = 4

N_BLK = 1024


def _keys_body(w_ref, xt_ref, out_ref, vpad_ref):
    xt = xt_ref[...]
    out_ref[...] = jnp.dot(w_ref[...], xt, preferred_element_type=jnp.float32)
    vpad_ref[:, :D] = xt.T


def _compute(w2, xt):
    return pl.pallas_call(
        _keys_body,
        grid=(pl.cdiv(N, N_BLK),),
        in_specs=[
            pl.BlockSpec((S * D_K, D), lambda i: (0, 0)),
            pl.BlockSpec((D, N_BLK), lambda i: (0, i)),
        ],
        out_specs=[
            pl.BlockSpec((S * D_K, N_BLK), lambda i: (0, i)),
            pl.BlockSpec((N_BLK, DPAD), lambda i: (i, 0)),
        ],
        out_shape=[
            jax.ShapeDtypeStruct((S * D_K, N), jnp.float32),
            jax.ShapeDtypeStruct((N, DPAD), jnp.float32),
        ],
    )(w2, xt)


@functools.lru_cache(maxsize=None)
def _make_gather():
    info = plsc.get_sparse_core_info()
    nw = info.num_cores * info.num_subcores
    b_per_w = K // nw
    mesh = plsc.VectorSubcoreMesh(core_axis_name="c", subcore_axis_name="s")

    @functools.partial(
        pl.kernel,
        mesh=mesh,
        out_type=jax.ShapeDtypeStruct((K, DPAD), jnp.float32),
        scratch_types=[
            pltpu.VMEM((b_per_w,), jnp.int32),
            pltpu.VMEM((b_per_w, DPAD), jnp.float32),
            pltpu.SemaphoreType.DMA,
        ],
    )
    def gather_k(table_hbm, idx_hbm, out_hbm, idx_v, rows_v, sem):
        wid = lax.axis_index("s") * info.num_cores + lax.axis_index("c")
        base = wid * b_per_w
        pltpu.sync_copy(idx_hbm.at[pl.ds(base, b_per_w)], idx_v)
        pltpu.async_copy(table_hbm.at[idx_v], rows_v, sem).wait()
        pltpu.sync_copy(rows_v, out_hbm.at[pl.ds(base, b_per_w)])

    return gather_k


def kernel(vectors, key_proj, indices):
    xt = vectors.T
    w2 = jnp.transpose(key_proj, (0, 2, 1)).reshape(S * D_K, D)
    out2, vpad = _compute(w2, xt)
    keys = out2.reshape(S, D_K, N).transpose(0, 2, 1)
    vecs = _make_gather()(vpad, indices)
    U = vecs[:, :S1].reshape(-1, D_B, R)
    V = vecs[:, S1:S2].reshape(-1, R, D_A)
    b = vecs[:, S2:S3]
    return keys, U, V, b


# speedup vs baseline: 1.0834x; 1.0834x over previous
import functools

import jax
import jax.numpy as jnp
from jax import lax
from jax.experimental import pallas as pl
from jax.experimental.pallas import tpu as pltpu
from jax.experimental.pallas import tpu_sc as plsc

N = 100000
D = 576
DPAD = 640
S = 4
D_K = 64
S1, S2, S3 = 256, 512, 576
K = 4096
D_B = 64
D_A = 64
R = 4

N_BLK = 4096


def _keys_body(w_ref, xt_ref, out_ref, vpad_ref):
    xt = xt_ref[...]
    out_ref[...] = jnp.dot(w_ref[...], xt, preferred_element_type=jnp.float32)
    vpad_ref[:, :D] = xt.T


def _compute(w2, xt):
    return pl.pallas_call(
        _keys_body,
        grid=(pl.cdiv(N, N_BLK),),
        in_specs=[
            pl.BlockSpec((S * D_K, D), lambda i: (0, 0)),
            pl.BlockSpec((D, N_BLK), lambda i: (0, i)),
        ],
        out_specs=[
            pl.BlockSpec((S * D_K, N_BLK), lambda i: (0, i)),
            pl.BlockSpec((N_BLK, DPAD), lambda i: (i, 0)),
        ],
        out_shape=[
            jax.ShapeDtypeStruct((S * D_K, N), jnp.float32),
            jax.ShapeDtypeStruct((N, DPAD), jnp.float32),
        ],
    )(w2, xt)


@functools.lru_cache(maxsize=None)
def _make_gather():
    info = plsc.get_sparse_core_info()
    nw = info.num_cores * info.num_subcores
    b_per_w = K // nw
    mesh = plsc.VectorSubcoreMesh(core_axis_name="c", subcore_axis_name="s")

    @functools.partial(
        pl.kernel,
        mesh=mesh,
        out_type=jax.ShapeDtypeStruct((K, DPAD), jnp.float32),
        scratch_types=[
            pltpu.VMEM((b_per_w,), jnp.int32),
            pltpu.VMEM((b_per_w, DPAD), jnp.float32),
            pltpu.SemaphoreType.DMA,
        ],
    )
    def gather_k(table_hbm, idx_hbm, out_hbm, idx_v, rows_v, sem):
        wid = lax.axis_index("s") * info.num_cores + lax.axis_index("c")
        base = wid * b_per_w
        pltpu.sync_copy(idx_hbm.at[pl.ds(base, b_per_w)], idx_v)
        pltpu.async_copy(table_hbm.at[idx_v], rows_v, sem).wait()
        pltpu.sync_copy(rows_v, out_hbm.at[pl.ds(base, b_per_w)])

    return gather_k


def kernel(vectors, key_proj, indices):
    xt = vectors.T
    w2 = jnp.transpose(key_proj, (0, 2, 1)).reshape(S * D_K, D)
    out2, vpad = _compute(w2, xt)
    keys = out2.reshape(S, D_K, N).transpose(0, 2, 1)
    vecs = _make_gather()(vpad, indices)
    U = vecs[:, :S1].reshape(-1, D_B, R)
    V = vecs[:, S1:S2].reshape(-1, R, D_A)
    b = vecs[:, S2:S3]
    return keys, U, V, b


# speedup vs baseline: 1.0892x; 1.0053x over previous
import functools

import jax
import jax.numpy as jnp
from jax import lax
from jax.experimental import pallas as pl
from jax.experimental.pallas import tpu as pltpu
from jax.experimental.pallas import tpu_sc as plsc

N = 100000
D = 576
DPAD = 640
S = 4
D_K = 64
S1, S2, S3 = 256, 512, 576
K = 4096
D_B = 64
D_A = 64
R = 4

N_BLK = 5120


def _keys_body(w_ref, xt_ref, out_ref, vpad_ref):
    xt = xt_ref[...]
    out_ref[...] = jnp.dot(w_ref[...], xt, preferred_element_type=jnp.float32)
    vpad_ref[:, :D] = xt.T


def _compute(w2, xt):
    return pl.pallas_call(
        _keys_body,
        grid=(pl.cdiv(N, N_BLK),),
        in_specs=[
            pl.BlockSpec((S * D_K, D), lambda i: (0, 0)),
            pl.BlockSpec((D, N_BLK), lambda i: (0, i)),
        ],
        out_specs=[
            pl.BlockSpec((S * D_K, N_BLK), lambda i: (0, i)),
            pl.BlockSpec((N_BLK, DPAD), lambda i: (i, 0)),
        ],
        out_shape=[
            jax.ShapeDtypeStruct((S * D_K, N), jnp.float32),
            jax.ShapeDtypeStruct((N, DPAD), jnp.float32),
        ],
    )(w2, xt)


@functools.lru_cache(maxsize=None)
def _make_gather():
    info = plsc.get_sparse_core_info()
    nw = info.num_cores * info.num_subcores
    b_per_w = K // nw
    mesh = plsc.VectorSubcoreMesh(core_axis_name="c", subcore_axis_name="s")

    @functools.partial(
        pl.kernel,
        mesh=mesh,
        out_type=jax.ShapeDtypeStruct((K, DPAD), jnp.float32),
        scratch_types=[
            pltpu.VMEM((b_per_w,), jnp.int32),
            pltpu.VMEM((b_per_w, DPAD), jnp.float32),
            pltpu.SemaphoreType.DMA,
        ],
    )
    def gather_k(table_hbm, idx_hbm, out_hbm, idx_v, rows_v, sem):
        wid = lax.axis_index("s") * info.num_cores + lax.axis_index("c")
        base = wid * b_per_w
        pltpu.sync_copy(idx_hbm.at[pl.ds(base, b_per_w)], idx_v)
        pltpu.async_copy(table_hbm.at[idx_v], rows_v, sem).wait()
        pltpu.sync_copy(rows_v, out_hbm.at[pl.ds(base, b_per_w)])

    return gather_k


def kernel(vectors, key_proj, indices):
    xt = vectors.T
    w2 = jnp.transpose(key_proj, (0, 2, 1)).reshape(S * D_K, D)
    out2, vpad = _compute(w2, xt)
    keys = out2.reshape(S, D_K, N).transpose(0, 2, 1)
    vecs = _make_gather()(vpad, indices)
    U = vecs[:, :S1].reshape(-1, D_B, R)
    V = vecs[:, S1:S2].reshape(-1, R, D_A)
    b = vecs[:, S2:S3]
    return keys, U, V, b
